# baseline (speedup 1.0000x reference)
import dataclasses
import functools

import jax
import jax.numpy as jnp
from jax import lax
from jax.experimental import pallas as pl
from jax.experimental.pallas import tpu as pltpu
from jax.experimental.pallas import tpu_sc as plsc

N = 10000
E = 320000
D = 128

NC = 2
NS = 16
NW = NC * NS
CHUNK = 128
CPB = 80
EP = NW * CPB * CHUNK
NSP = 10240
STAGE = 16
NSTAGE = CPB // STAGE
RC = 80
NRCH = N // RC
RFULL = NRCH // NS
RTAIL = NRCH - RFULL * NS
CNTW = 4
NP = 10240
CROWS = NP // 32
CSUB = 64
L = 16


def _sc_agg(x, srcr, dstr, zd, with_cnt):
    mesh = plsc.VectorSubcoreMesh(core_axis_name="c", subcore_axis_name="s")
    NR = N + CROWS if with_cnt else N
    out_type = jax.ShapeDtypeStruct((NC, NR, D), jnp.float32)
    scratch = [
        pltpu.VMEM((STAGE, CHUNK), jnp.int32),
        pltpu.VMEM((STAGE, CHUNK), jnp.int32),
        pltpu.VMEM((CSUB,), jnp.int32),
        pltpu.VMEM((CHUNK,), jnp.int32),
        pltpu.VMEM((CHUNK,), jnp.int32),
        pltpu.VMEM((CHUNK,), jnp.int32),
        pltpu.VMEM((CHUNK,), jnp.int32),
        pltpu.VMEM((CHUNK, D), jnp.float32),
        pltpu.VMEM((CHUNK, D), jnp.float32),
        pltpu.VMEM((CSUB, D), jnp.float32),
        pltpu.VMEM_SHARED((NSP, D), jnp.float32),
        pltpu.VMEM_SHARED((CROWS, D), jnp.float32),
        pltpu.SemaphoreType.DMA,
        pltpu.SemaphoreType.DMA,
    ]

    def body(x_hbm, srcr_hbm, dstr_hbm, zd_hbm, agg_out, *rest):
        (srcv, dstv, ridxv, src1_0, src1_1, dst1_0, dst1_1,
         rows0, rows1, crowsv, agg_sh, cnt_sh, sem0, sem1) = rest
        rows = (rows0, rows1)
        src1 = (src1_0, src1_1)
        dst1 = (dst1_0, dst1_1)
        sems = (sem0, sem1)
        cid = lax.axis_index("c")
        sid = lax.axis_index("s")
        wid = sid * NC + cid

        def row_chunks(fn):
            @pl.loop(0, RFULL)
            def _(k):
                fn(pl.multiple_of((k * NS + sid) * RC, 8))

            @pl.when(sid < RTAIL)
            def _():
                fn(pl.multiple_of((RFULL * NS + sid) * RC, 8))

        row_chunks(lambda off: pltpu.sync_copy(zd_hbm, agg_sh.at[pl.ds(off, RC)]))
        if with_cnt:
            @pl.when(sid < CROWS // RC)
            def _():
                pltpu.sync_copy(
                    zd_hbm, cnt_sh.at[pl.ds(pl.multiple_of(sid * RC, 8), RC)])

            zv = jnp.zeros((L,), jnp.float32)

            @pl.loop(0, CSUB)
            def _(r):
                for j in range(D // L):
                    crowsv[r, pl.ds(j * L, L)] = zv
        plsc.subcore_barrier()

        lanes = lax.iota(jnp.int32, L)
        onev = jnp.full((L,), 1.0, jnp.float32)
        zerov = jnp.zeros((L,), jnp.float32)

        def start(k, b):
            for j in range(CHUNK // L):
                src1[b][pl.ds(j * L, L)] = srcv[k, pl.ds(j * L, L)]
                dst1[b][pl.ds(j * L, L)] = dstv[k, pl.ds(j * L, L)]
            pltpu.async_copy(x_hbm.at[src1[b]], rows[b], sems[b])

        def cnt_work(k):
            if with_cnt:
                for h in range(CHUNK // CSUB):
                    for j in range(CSUB // L):
                        d = dstv[k, pl.ds(h * CSUB + j * L, L)]
                        ridxv[pl.ds(j * L, L)] = lax.shift_right_logical(d, 5)
                        plsc.store_scatter(
                            crowsv, [j * L + lanes, (d & 31) * CNTW], onev)
                    pltpu.sync_copy(crowsv, cnt_sh.at[ridxv], add=True)
                    for j in range(CSUB // L):
                        d = dstv[k, pl.ds(h * CSUB + j * L, L)]
                        plsc.store_scatter(
                            crowsv, [j * L + lanes, (d & 31) * CNTW], zerov)

        def finish(k, b):
            cnt_work(k)
            pltpu.make_async_copy(x_hbm.at[src1[b]], rows[b], sems[b]).wait()
            pltpu.sync_copy(rows[b], agg_sh.at[dst1[b]], add=True)

        def chunk_sync(k):
            for j in range(CHUNK // L):
                src1[0][pl.ds(j * L, L)] = srcv[k, pl.ds(j * L, L)]
                dst1[0][pl.ds(j * L, L)] = dstv[k, pl.ds(j * L, L)]
            cp = pltpu.async_copy(x_hbm.at[src1[0]], rows[0], sems[0])
            cnt_work(k)
            cp.wait()
            pltpu.sync_copy(rows[0], agg_sh.at[dst1[0]], add=True)

        @pl.loop(0, NSTAGE)
        def _(s):
            off = pl.multiple_of(s * STAGE, 8)
            pltpu.sync_copy(srcr_hbm.at[wid, pl.ds(off, STAGE)], srcv)
            pltpu.sync_copy(dstr_hbm.at[wid, pl.ds(off, STAGE)], dstv)

            @pl.loop(0, STAGE)
            def _(k):
                chunk_sync(k)

        plsc.subcore_barrier()

        row_chunks(lambda off: pltpu.sync_copy(
            agg_sh.at[pl.ds(off, RC)], agg_out.at[cid, pl.ds(off, RC)]))
        if with_cnt:
            @pl.when(sid < CROWS // RC)
            def _():
                off = pl.multiple_of(sid * RC, 8)
                pltpu.sync_copy(cnt_sh.at[pl.ds(off, RC)],
                                agg_out.at[cid, pl.ds(N + off, RC)])

    cp = pltpu.CompilerParams()
    if "needs_layout_passes" in pltpu.CompilerParams.__dataclass_fields__:
        cp = dataclasses.replace(cp, needs_layout_passes=False)
    kern = pl.kernel(body, out_type=out_type, mesh=mesh,
                     scratch_types=scratch, compiler_params=cp)
    return kern(x, srcr, dstr, zd)


BLK = 400


def _tc_dense(aggp, cntp, x, Wl, Wr, b, a):
    def body(aggp_ref, cntp_ref, x_ref, wl_ref, wr_ref, b_ref, a_ref, o_ref):
        s = aggp_ref[0] + aggp_ref[1]
        c = cntp_ref[0, :, 0:1] + cntp_ref[1, :, 0:1]
        agg = s / jnp.maximum(c, 1.0)
        h = (jnp.dot(agg, wl_ref[...], preferred_element_type=jnp.float32)
             + jnp.dot(x_ref[...], wr_ref[...], preferred_element_type=jnp.float32)
             + b_ref[...])
        o_ref[...] = jnp.where(h >= 0, h, a_ref[...] * h)

    return pl.pallas_call(
        body,
        grid=(N // BLK,),
        in_specs=[
            pl.BlockSpec((NC, BLK, D), lambda i: (0, i, 0)),
            pl.BlockSpec((NC, BLK, CNTW), lambda i: (0, i, 0)),
            pl.BlockSpec((BLK, D), lambda i: (i, 0)),
            pl.BlockSpec((D, D), lambda i: (0, 0)),
            pl.BlockSpec((D, D), lambda i: (0, 0)),
            pl.BlockSpec((1, D), lambda i: (0, 0)),
            pl.BlockSpec((1, D), lambda i: (0, 0)),
        ],
        out_specs=pl.BlockSpec((BLK, D), lambda i: (i, 0)),
        out_shape=jax.ShapeDtypeStruct((N, D), jnp.float32),
    )(aggp, cntp, x, Wl, Wr, b.reshape(1, D), a.reshape(1, D))


def kernel(x, edge_index, Wl1, Wr1, b1, a1, Wl2, Wr2, b2, a2):
    src = jnp.concatenate(
        [edge_index[0], jnp.zeros((EP - E,), edge_index.dtype)]
    ).reshape(NW, CPB, CHUNK)
    dst = jnp.concatenate(
        [edge_index[1],
         N + jnp.arange(EP - E, dtype=edge_index.dtype) % (NSP - N)]
    ).reshape(NW, CPB, CHUNK)
    zd = jnp.zeros((RC, D), jnp.float32)

    aggcnt = _sc_agg(x, src, dst, zd, with_cnt=True)
    agg1p = aggcnt[:, :N]
    cntp = aggcnt[:, N:].reshape(NC, NP, CNTW)[:, :N]
    h = _tc_dense(agg1p, cntp, x, Wl1, Wr1, b1, a1)
    agg2p = _sc_agg(h, src, dst, zd, with_cnt=False)
    return _tc_dense(agg2p, cntp, h, Wl2, Wr2, b2, a2)


# speedup vs baseline: 2.3137x; 2.3137x over previous
import dataclasses

import jax
import jax.numpy as jnp
from jax import lax
from jax.experimental import pallas as pl
from jax.experimental.pallas import tpu as pltpu
from jax.experimental.pallas import tpu_sc as plsc

N = 10000
E = 320000
D = 128

NC = 2
NS = 16
NW = NC * NS
CHUNK = 128
CHUNKS = E // CHUNK
FULL = CHUNKS // NW
TAIL = CHUNKS - FULL * NW
RC = 80
NRCH = N // RC
RFULL = NRCH // NS
RTAIL = NRCH - RFULL * NS
CNTW = 4
NP = 10240
CROWS = NP // 32
CSUB = 64
L = 16


def _sc_agg(x, src, dst, zd, with_cnt):
    mesh = plsc.VectorSubcoreMesh(core_axis_name="c", subcore_axis_name="s")
    NR = N + CROWS if with_cnt else N
    out_type = jax.ShapeDtypeStruct((NC, NR, D), jnp.float32)
    scratch = [
        pltpu.VMEM((CHUNK,), jnp.int32),
        pltpu.VMEM((CHUNK,), jnp.int32),
        pltpu.VMEM((CHUNK,), jnp.int32),
        pltpu.VMEM((CHUNK,), jnp.int32),
        pltpu.VMEM((CSUB,), jnp.int32),
        pltpu.VMEM((CHUNK, D), jnp.float32),
        pltpu.VMEM((CHUNK, D), jnp.float32),
        pltpu.VMEM((CSUB, D), jnp.float32),
        pltpu.VMEM_SHARED((N, D), jnp.float32),
        pltpu.VMEM_SHARED((CROWS, D), jnp.float32),
        pltpu.SemaphoreType.DMA,
        pltpu.SemaphoreType.DMA,
    ]

    def body(x_hbm, src_hbm, dst_hbm, zd_hbm, agg_out, *rest):
        (srcv0, srcv1, dstv0, dstv1, ridxv, rows0, rows1, crowsv,
         agg_sh, cnt_sh, sem0, sem1) = rest
        srcv = (srcv0, srcv1)
        dstv = (dstv0, dstv1)
        rows = (rows0, rows1)
        sems = (sem0, sem1)
        cid = lax.axis_index("c")
        sid = lax.axis_index("s")
        wid = sid * NC + cid

        def row_chunks(fn):
            @pl.loop(0, RFULL)
            def _(k):
                fn(pl.multiple_of((k * NS + sid) * RC, 8))

            @pl.when(sid < RTAIL)
            def _():
                fn(pl.multiple_of((RFULL * NS + sid) * RC, 8))

        row_chunks(lambda off: pltpu.sync_copy(zd_hbm, agg_sh.at[pl.ds(off, RC)]))
        if with_cnt:
            @pl.when(sid < CROWS // RC)
            def _():
                pltpu.sync_copy(
                    zd_hbm, cnt_sh.at[pl.ds(pl.multiple_of(sid * RC, 8), RC)])

            zv = jnp.zeros((L,), jnp.float32)

            @pl.loop(0, CSUB)
            def _(r):
                for j in range(D // L):
                    crowsv[r, pl.ds(j * L, L)] = zv
        plsc.subcore_barrier()

        lanes = lax.iota(jnp.int32, L)
        onev = jnp.full((L,), 1.0, jnp.float32)
        zerov = jnp.zeros((L,), jnp.float32)

        def cnt_work(dv):
            if not with_cnt:
                return
            for h in range(CHUNK // CSUB):
                for j in range(CSUB // L):
                    d = dv[pl.ds(h * CSUB + j * L, L)]
                    ridxv[pl.ds(j * L, L)] = lax.shift_right_logical(d, 5)
                    plsc.store_scatter(
                        crowsv, [j * L + lanes, (d & 31) * CNTW], onev)
                pltpu.sync_copy(crowsv, cnt_sh.at[ridxv], add=True)
                for j in range(CSUB // L):
                    d = dv[pl.ds(h * CSUB + j * L, L)]
                    plsc.store_scatter(
                        crowsv, [j * L + lanes, (d & 31) * CNTW], zerov)

        def start(c, b):
            base = pl.multiple_of(c * CHUNK, CHUNK)
            pltpu.sync_copy(src_hbm.at[pl.ds(base, CHUNK)], srcv[b])
            pltpu.sync_copy(dst_hbm.at[pl.ds(base, CHUNK)], dstv[b])
            pltpu.async_copy(x_hbm.at[srcv[b]], rows[b], sems[b])

        def finish(b):
            cnt_work(dstv[b])
            pltpu.make_async_copy(x_hbm.at[srcv[b]], rows[b], sems[b]).wait()
            pltpu.sync_copy(rows[b], agg_sh.at[dstv[b]], add=True)

        start(wid, 0)

        @pl.loop(0, FULL // 2 - 1)
        def _(t):
            start((2 * t + 1) * NW + wid, 1)
            finish(0)
            start((2 * t + 2) * NW + wid, 0)
            finish(1)

        start((FULL - 1) * NW + wid, 1)
        finish(0)
        finish(1)

        @pl.when(wid < TAIL)
        def _():
            start(FULL * NW + wid, 0)
            finish(0)

        plsc.subcore_barrier()

        row_chunks(lambda off: pltpu.sync_copy(
            agg_sh.at[pl.ds(off, RC)], agg_out.at[cid, pl.ds(off, RC)]))
        if with_cnt:
            @pl.when(sid < CROWS // RC)
            def _():
                off = pl.multiple_of(sid * RC, 8)
                pltpu.sync_copy(cnt_sh.at[pl.ds(off, RC)],
                                agg_out.at[cid, pl.ds(N + off, RC)])

    cp = pltpu.CompilerParams()
    if "needs_layout_passes" in pltpu.CompilerParams.__dataclass_fields__:
        cp = dataclasses.replace(cp, needs_layout_passes=False)
    kern = pl.kernel(body, out_type=out_type, mesh=mesh,
                     scratch_types=scratch, compiler_params=cp)
    return kern(x, src, dst, zd)


BLK = 400


def _tc_dense(aggp, cntp, x, Wl, Wr, b, a):
    def body(aggp_ref, cntp_ref, x_ref, wl_ref, wr_ref, b_ref, a_ref, o_ref):
        s = aggp_ref[0] + aggp_ref[1]
        c = cntp_ref[0, :, 0:1] + cntp_ref[1, :, 0:1]
        agg = s / jnp.maximum(c, 1.0)
        h = (jnp.dot(agg, wl_ref[...], preferred_element_type=jnp.float32)
             + jnp.dot(x_ref[...], wr_ref[...], preferred_element_type=jnp.float32)
             + b_ref[...])
        o_ref[...] = jnp.where(h >= 0, h, a_ref[...] * h)

    return pl.pallas_call(
        body,
        grid=(N // BLK,),
        in_specs=[
            pl.BlockSpec((NC, BLK, D), lambda i: (0, i, 0)),
            pl.BlockSpec((NC, BLK, CNTW), lambda i: (0, i, 0)),
            pl.BlockSpec((BLK, D), lambda i: (i, 0)),
            pl.BlockSpec((D, D), lambda i: (0, 0)),
            pl.BlockSpec((D, D), lambda i: (0, 0)),
            pl.BlockSpec((1, D), lambda i: (0, 0)),
            pl.BlockSpec((1, D), lambda i: (0, 0)),
        ],
        out_specs=pl.BlockSpec((BLK, D), lambda i: (i, 0)),
        out_shape=jax.ShapeDtypeStruct((N, D), jnp.float32),
    )(aggp, cntp, x, Wl, Wr, b.reshape(1, D), a.reshape(1, D))


def kernel(x, edge_index, Wl1, Wr1, b1, a1, Wl2, Wr2, b2, a2):
    src = edge_index[0]
    dst = edge_index[1]
    zd = jnp.zeros((RC, D), jnp.float32)

    aggcnt = _sc_agg(x, src, dst, zd, with_cnt=True)
    agg1p = aggcnt[:, :N]
    cntp = aggcnt[:, N:].reshape(NC, NP, CNTW)[:, :N]
    h = _tc_dense(agg1p, cntp, x, Wl1, Wr1, b1, a1)
    agg2p = _sc_agg(h, src, dst, zd, with_cnt=False)
    return _tc_dense(agg2p, cntp, h, Wl2, Wr2, b2, a2)


# speedup vs baseline: 2.5248x; 1.0913x over previous
import dataclasses

import jax
import jax.numpy as jnp
from jax import lax
from jax.experimental import pallas as pl
from jax.experimental.pallas import tpu as pltpu
from jax.experimental.pallas import tpu_sc as plsc

N = 10000
E = 320000
D = 128

NC = 2
NS = 16
NW = NC * NS
CHUNK = 128
CHUNKS = E // CHUNK
FULL = CHUNKS // NW
TAIL = CHUNKS - FULL * NW
RC = 80
NRCH = N // RC
RFULL = NRCH // NS
RTAIL = NRCH - RFULL * NS
CNTW = 4
NP = 10240
CROWS = NP // 32
CSUB = 64
L = 16


def _sc_agg(x, src, dst, zd, with_cnt):
    mesh = plsc.VectorSubcoreMesh(core_axis_name="c", subcore_axis_name="s")
    NR = N + CROWS if with_cnt else N
    out_type = jax.ShapeDtypeStruct((NC, NR, D), jnp.float32)
    scratch = [
        pltpu.VMEM((CHUNK,), jnp.int32),
        pltpu.VMEM((CHUNK,), jnp.int32),
        pltpu.VMEM((CHUNK,), jnp.int32),
        pltpu.VMEM((CHUNK,), jnp.int32),
        pltpu.VMEM((CSUB,), jnp.int32),
        pltpu.VMEM((CHUNK, D), jnp.float32),
        pltpu.VMEM((CHUNK, D), jnp.float32),
        pltpu.VMEM((CSUB, D), jnp.float32),
        pltpu.VMEM_SHARED((N, D), jnp.float32),
        pltpu.VMEM_SHARED((CROWS, D), jnp.float32),
        pltpu.SemaphoreType.DMA,
        pltpu.SemaphoreType.DMA,
        pltpu.SemaphoreType.DMA,
        pltpu.SemaphoreType.DMA,
    ]

    def body(x_hbm, src_hbm, dst_hbm, zd_hbm, agg_out, *rest):
        (srcv0, srcv1, dstv0, dstv1, ridxv, rows0, rows1, crowsv,
         agg_sh, cnt_sh, sem0, sem1, isem0, isem1) = rest
        srcv = (srcv0, srcv1)
        dstv = (dstv0, dstv1)
        rows = (rows0, rows1)
        sems = (sem0, sem1)
        isems = (isem0, isem1)
        cid = lax.axis_index("c")
        sid = lax.axis_index("s")
        wid = sid * NC + cid

        def row_chunks(fn):
            @pl.loop(0, RFULL)
            def _(k):
                fn(pl.multiple_of((k * NS + sid) * RC, 8))

            @pl.when(sid < RTAIL)
            def _():
                fn(pl.multiple_of((RFULL * NS + sid) * RC, 8))

        row_chunks(lambda off: pltpu.sync_copy(zd_hbm, agg_sh.at[pl.ds(off, RC)]))
        if with_cnt:
            @pl.when(sid < CROWS // RC)
            def _():
                pltpu.sync_copy(
                    zd_hbm, cnt_sh.at[pl.ds(pl.multiple_of(sid * RC, 8), RC)])

            zv = jnp.zeros((L,), jnp.float32)

            @pl.loop(0, CSUB)
            def _(r):
                for j in range(D // L):
                    crowsv[r, pl.ds(j * L, L)] = zv
        plsc.subcore_barrier()

        lanes = lax.iota(jnp.int32, L)
        onev = jnp.full((L,), 1.0, jnp.float32)
        zerov = jnp.zeros((L,), jnp.float32)

        def cnt_work(dv):
            if not with_cnt:
                return
            for h in range(CHUNK // CSUB):
                for j in range(CSUB // L):
                    d = dv[pl.ds(h * CSUB + j * L, L)]
                    ridxv[pl.ds(j * L, L)] = lax.shift_right_logical(d, 5)
                    plsc.store_scatter(
                        crowsv, [j * L + lanes, (d & 31) * CNTW], onev)
                pltpu.sync_copy(crowsv, cnt_sh.at[ridxv], add=True)
                for j in range(CSUB // L):
                    d = dv[pl.ds(h * CSUB + j * L, L)]
                    plsc.store_scatter(
                        crowsv, [j * L + lanes, (d & 31) * CNTW], zerov)

        def prefetch(c, b):
            base = pl.multiple_of(c * CHUNK, CHUNK)
            pltpu.async_copy(src_hbm.at[pl.ds(base, CHUNK)], srcv[b], isems[b])
            pltpu.async_copy(dst_hbm.at[pl.ds(base, CHUNK)], dstv[b], isems[b])

        def start(c, b):
            base = pl.multiple_of(c * CHUNK, CHUNK)
            pltpu.make_async_copy(
                src_hbm.at[pl.ds(base, CHUNK)], srcv[b], isems[b]).wait()
            pltpu.make_async_copy(
                dst_hbm.at[pl.ds(base, CHUNK)], dstv[b], isems[b]).wait()
            pltpu.async_copy(x_hbm.at[srcv[b]], rows[b], sems[b])

        def finish(b):
            cnt_work(dstv[b])
            pltpu.make_async_copy(x_hbm.at[srcv[b]], rows[b], sems[b]).wait()
            pltpu.sync_copy(rows[b], agg_sh.at[dstv[b]], add=True)

        prefetch(wid, 0)
        prefetch(NW + wid, 1)
        start(wid, 0)

        @pl.loop(0, FULL // 2 - 1)
        def _(t):
            start((2 * t + 1) * NW + wid, 1)
            finish(0)
            prefetch((2 * t + 2) * NW + wid, 0)
            finish(1)
            prefetch((2 * t + 3) * NW + wid, 1)
            start((2 * t + 2) * NW + wid, 0)

        start((FULL - 1) * NW + wid, 1)
        finish(0)
        finish(1)

        @pl.when(wid < TAIL)
        def _():
            prefetch(FULL * NW + wid, 0)
            start(FULL * NW + wid, 0)
            finish(0)

        plsc.subcore_barrier()

        row_chunks(lambda off: pltpu.sync_copy(
            agg_sh.at[pl.ds(off, RC)], agg_out.at[cid, pl.ds(off, RC)]))
        if with_cnt:
            @pl.when(sid < CROWS // RC)
            def _():
                off = pl.multiple_of(sid * RC, 8)
                pltpu.sync_copy(cnt_sh.at[pl.ds(off, RC)],
                                agg_out.at[cid, pl.ds(N + off, RC)])

    cp = pltpu.CompilerParams()
    if "needs_layout_passes" in pltpu.CompilerParams.__dataclass_fields__:
        cp = dataclasses.replace(cp, needs_layout_passes=False)
    kern = pl.kernel(body, out_type=out_type, mesh=mesh,
                     scratch_types=scratch, compiler_params=cp)
    return kern(x, src, dst, zd)


BLK = 400


def _tc_dense(aggp, cntp, x, Wl, Wr, b, a):
    def body(aggp_ref, cntp_ref, x_ref, wl_ref, wr_ref, b_ref, a_ref, o_ref):
        s = aggp_ref[0] + aggp_ref[1]
        c = cntp_ref[0, :, 0:1] + cntp_ref[1, :, 0:1]
        agg = s / jnp.maximum(c, 1.0)
        h = (jnp.dot(agg, wl_ref[...], preferred_element_type=jnp.float32)
             + jnp.dot(x_ref[...], wr_ref[...], preferred_element_type=jnp.float32)
             + b_ref[...])
        o_ref[...] = jnp.where(h >= 0, h, a_ref[...] * h)

    return pl.pallas_call(
        body,
        grid=(N // BLK,),
        in_specs=[
            pl.BlockSpec((NC, BLK, D), lambda i: (0, i, 0)),
            pl.BlockSpec((NC, BLK, CNTW), lambda i: (0, i, 0)),
            pl.BlockSpec((BLK, D), lambda i: (i, 0)),
            pl.BlockSpec((D, D), lambda i: (0, 0)),
            pl.BlockSpec((D, D), lambda i: (0, 0)),
            pl.BlockSpec((1, D), lambda i: (0, 0)),
            pl.BlockSpec((1, D), lambda i: (0, 0)),
        ],
        out_specs=pl.BlockSpec((BLK, D), lambda i: (i, 0)),
        out_shape=jax.ShapeDtypeStruct((N, D), jnp.float32),
    )(aggp, cntp, x, Wl, Wr, b.reshape(1, D), a.reshape(1, D))


def kernel(x, edge_index, Wl1, Wr1, b1, a1, Wl2, Wr2, b2, a2):
    src = edge_index[0]
    dst = edge_index[1]
    zd = jnp.zeros((RC, D), jnp.float32)

    aggcnt = _sc_agg(x, src, dst, zd, with_cnt=True)
    agg1p = aggcnt[:, :N]
    cntp = aggcnt[:, N:].reshape(NC, NP, CNTW)[:, :N]
    h = _tc_dense(agg1p, cntp, x, Wl1, Wr1, b1, a1)
    agg2p = _sc_agg(h, src, dst, zd, with_cnt=False)
    return _tc_dense(agg2p, cntp, h, Wl2, Wr2, b2, a2)
